# Initial kernel scaffold; baseline (speedup 1.0000x reference)
import jax
import jax.numpy as jnp
from jax.experimental import pallas as pl


def kernel(x, position_indices, embedding):
    raise NotImplementedError("write your pallas kernel here")


# speedup vs baseline: 2.0279x; 2.0279x over previous
import functools

import jax
import jax.numpy as jnp
from jax import lax
from jax.experimental import pallas as pl
from jax.experimental.pallas import tpu as pltpu
from jax.experimental.pallas import tpu_sc as plsc

EMBED_DIM = 64
NUM_WORKERS = 32
CHUNK = 128
LANES = 16


def _pos_embed_body(x_hbm, idx_hbm, tab_hbm, out_hbm, idx_v, xb, rows, sem):
    nc = 2
    wid = lax.axis_index("s") * nc + lax.axis_index("c")
    tok_per_worker = x_hbm.shape[0] // NUM_WORKERS
    n_chunks = tok_per_worker // CHUNK
    worker_base = wid * tok_per_worker

    @pl.loop(0, n_chunks)
    def _chunk(i):
        base = worker_base + i * CHUNK
        pltpu.sync_copy(idx_hbm.at[pl.ds(base, CHUNK)], idx_v)
        gather = pltpu.async_copy(tab_hbm.at[idx_v], rows, sem)
        pltpu.sync_copy(x_hbm.at[pl.ds(base, CHUNK)], xb)
        gather.wait()

        @pl.loop(0, CHUNK)
        def _row(r):
            for c in range(EMBED_DIM // LANES):
                sl = pl.ds(c * LANES, LANES)
                xb[r, sl] += rows[r, sl]

        pltpu.sync_copy(xb, out_hbm.at[pl.ds(base, CHUNK)])


@functools.partial(jax.jit, static_argnames=())
def kernel(x, position_indices, embedding):
    b, s, d = x.shape
    n = b * s
    x_flat = x.reshape(n, d)
    idx_flat = position_indices.reshape(n).astype(jnp.int32)

    mesh = plsc.VectorSubcoreMesh(
        core_axis_name="c", subcore_axis_name="s", num_cores=2, num_subcores=16
    )
    out = pl.kernel(
        _pos_embed_body,
        out_type=jax.ShapeDtypeStruct((n, d), x.dtype),
        mesh=mesh,
        scratch_types=[
            pltpu.VMEM((CHUNK,), jnp.int32),
            pltpu.VMEM((CHUNK, d), jnp.float32),
            pltpu.VMEM((CHUNK, d), jnp.float32),
            pltpu.SemaphoreType.DMA,
        ],
        compiler_params=pltpu.CompilerParams(use_tc_tiling_on_sc=False),
    )(x_flat, idx_flat, embedding)
    return out.reshape(b, s, d)
